# baseline (speedup 1.0000x reference)
import jax
import jax.numpy as jnp
from jax.experimental import pallas as pl


def _copy_body(emb_ref, out_ref):
    out_ref[...] = emb_ref[...]


def kernel(inputs, embedding):
    del inputs
    return pl.pallas_call(
        _copy_body,
        out_shape=jax.ShapeDtypeStruct((1, 128), jnp.float32),
    )(embedding)


# speedup vs baseline: 1.1407x; 1.1407x over previous
import jax
import jax.numpy as jnp
from jax.experimental import pallas as pl
from jax.experimental.pallas import tpu as pltpu


def _copy_body(emb_hbm, out_hbm, sem):
    cp = pltpu.make_async_copy(emb_hbm, out_hbm, sem)
    cp.start()
    cp.wait()


def kernel(inputs, embedding):
    del inputs
    return pl.pallas_call(
        _copy_body,
        in_specs=[pl.BlockSpec(memory_space=pl.ANY)],
        out_specs=pl.BlockSpec(memory_space=pl.ANY),
        out_shape=jax.ShapeDtypeStruct((1, 128), jnp.float32),
        scratch_shapes=[pltpu.SemaphoreType.DMA],
    )(embedding)
